# baseline (speedup 1.0000x reference)
import functools

import jax
import jax.numpy as jnp
from jax.experimental import pallas as pl
from jax.experimental.pallas import tpu as pltpu


def _body(uv0_ref, uv1_ref, vu0_ref, vu1_ref, u0_ref,
          w1u_ref, b1u_ref, w2u_ref, b2u_ref,
          w1v_ref, b1v_ref, w2v_ref, b2v_ref,
          o_ref, uvbf_ref, v1_ref, u1_ref, acc_ref, *, bm, nb, kh):
    f32 = jnp.float32
    bf = jnp.bfloat16
    p = pl.program_id(0)
    i = pl.program_id(1)
    prev = jnp.maximum(i - 1, 0)
    cur = jnp.minimum(i, nb - 1)

    def epilogue(w1_ref, b1_ref, w2_ref, b2_ref):
        aggb = acc_ref[...].astype(bf)
        h = jnp.dot(aggb, w1_ref[...], preferred_element_type=f32) + b1_ref[...]
        h = jnp.where(h > 0, h, 0.01 * h)
        hb = h.astype(bf)
        o = jnp.dot(hb, w2_ref[...], preferred_element_type=f32) + b2_ref[...]
        return jnp.where(o > 0, o, 0.01 * o)

    @pl.when(p == 0)
    def _phase0():
        @pl.when(i > 0)
        def _():
            out = epilogue(w1u_ref, b1u_ref, w2u_ref, b2u_ref)
            v1_ref[pl.ds(prev * bm, bm), :] = out.astype(bf)

        @pl.when(i < nb)
        def _():
            a0 = uv0_ref[...].astype(bf)
            a1 = uv1_ref[...].astype(bf)
            uvbf_ref[pl.ds(cur * bm, bm), :kh] = a0
            uvbf_ref[pl.ds(cur * bm, bm), kh:] = a1
            acc_ref[...] = (
                jnp.dot(a0, u0_ref[:kh, :], preferred_element_type=f32)
                + jnp.dot(a1, u0_ref[kh:, :], preferred_element_type=f32))

    @pl.when(p == 1)
    def _phase1():
        @pl.when(i > 0)
        def _():
            out = epilogue(w1v_ref, b1v_ref, w2v_ref, b2v_ref)
            u1_ref[pl.ds(prev * bm, bm), :] = out.astype(bf)

        @pl.when(i < nb)
        def _():
            a0 = vu0_ref[...].astype(bf)
            a1 = vu1_ref[...].astype(bf)
            acc_ref[...] = (
                jnp.dot(a0, v1_ref[:kh, :], preferred_element_type=f32)
                + jnp.dot(a1, v1_ref[kh:, :], preferred_element_type=f32))

    @pl.when(p == 2)
    def _phase2():
        @pl.when(i > 0)
        def _():
            o_ref[...] = epilogue(w1u_ref, b1u_ref, w2u_ref, b2u_ref)

        @pl.when(i < nb)
        def _():
            ab = uvbf_ref[pl.ds(cur * bm, bm), :]
            acc_ref[...] = jnp.dot(ab, u1_ref[...],
                                   preferred_element_type=f32)


def kernel(u_node_feats, v_node_feats, uv_adj_mat, vu_adj_mat,
           W1_uv, b1_uv, W2_uv, b2_uv, W1_vu, b1_vu, W2_vu, b2_vu):
    bf = jnp.bfloat16
    bm = 256
    n, k = uv_adj_mat.shape
    d = u_node_feats.shape[1]
    kh = k // 2
    nb = n // bm

    u0 = u_node_feats.astype(bf)
    w1u = W1_uv.astype(bf)
    w2u = W2_uv.astype(bf)
    w1v = W1_vu.astype(bf)
    w2v = W2_vu.astype(bf)
    b1u = b1_uv.reshape(1, d)
    b2u = b2_uv.reshape(1, d)
    b1v = b1_vu.reshape(1, d)
    b2v = b2_vu.reshape(1, d)

    uv_idx = lambda p, i: (jnp.where(p == 0, jnp.minimum(i, nb - 1), nb - 1), 0)
    uv_idx1 = lambda p, i: (jnp.where(p == 0, jnp.minimum(i, nb - 1), nb - 1), 1)
    vu_idx = lambda p, i: (jnp.where(p == 1, jnp.minimum(i, nb - 1), 0), 0)
    vu_idx1 = lambda p, i: (jnp.where(p == 1, jnp.minimum(i, nb - 1), 0), 1)
    const = lambda p, i: (0, 0)

    return pl.pallas_call(
        functools.partial(_body, bm=bm, nb=nb, kh=kh),
        grid=(3, nb + 1),
        in_specs=[
            pl.BlockSpec((bm, kh), uv_idx),
            pl.BlockSpec((bm, kh), uv_idx1),
            pl.BlockSpec((bm, kh), vu_idx),
            pl.BlockSpec((bm, kh), vu_idx1),
            pl.BlockSpec((k, d), const),
            pl.BlockSpec((d, d), const),
            pl.BlockSpec((1, d), const),
            pl.BlockSpec((d, d), const),
            pl.BlockSpec((1, d), const),
            pl.BlockSpec((d, d), const),
            pl.BlockSpec((1, d), const),
            pl.BlockSpec((d, d), const),
            pl.BlockSpec((1, d), const),
        ],
        out_specs=pl.BlockSpec(
            (bm, d), lambda p, i: (jnp.where(p == 2, jnp.maximum(i - 1, 0), 0), 0)),
        out_shape=jax.ShapeDtypeStruct((n, d), jnp.float32),
        scratch_shapes=[
            pltpu.VMEM((n, k), bf),
            pltpu.VMEM((n, d), bf),
            pltpu.VMEM((n, d), bf),
            pltpu.VMEM((bm, d), jnp.float32),
        ],
        compiler_params=pltpu.CompilerParams(
            dimension_semantics=("arbitrary", "arbitrary"),
            vmem_limit_bytes=110 * 1024 * 1024,
        ),
    )(uv_adj_mat, uv_adj_mat, vu_adj_mat, vu_adj_mat, u0,
      w1u, b1u, w2u, b2u, w1v, b1v, w2v, b2v)


# speedup vs baseline: 1.0223x; 1.0223x over previous
import functools

import jax
import jax.numpy as jnp
from jax.experimental import pallas as pl
from jax.experimental.pallas import tpu as pltpu


def _body(uv0_ref, uv1_ref, vu0_ref, vu1_ref, u0_ref,
          w1u_ref, b1u_ref, w2u_ref, b2u_ref,
          w1v_ref, b1v_ref, w2v_ref, b2v_ref,
          o_ref, uvbf_ref, v1_ref, u1_ref, acc_ref, *, bm, nb, kh):
    f32 = jnp.float32
    bf = jnp.bfloat16
    p = pl.program_id(0)
    i = pl.program_id(1)
    prev = jnp.maximum(i - 1, 0)
    cur = jnp.minimum(i, nb - 1)

    def epilogue(w1_ref, b1_ref, w2_ref, b2_ref):
        aggb = acc_ref[...].astype(bf)
        h = jnp.dot(aggb, w1_ref[...], preferred_element_type=f32) + b1_ref[...]
        h = jnp.where(h > 0, h, 0.01 * h)
        hb = h.astype(bf)
        o = jnp.dot(hb, w2_ref[...], preferred_element_type=f32) + b2_ref[...]
        return jnp.where(o > 0, o, 0.01 * o)

    @pl.when(p == 0)
    def _phase0():
        out = epilogue(w1u_ref, b1u_ref, w2u_ref, b2u_ref)
        v1_ref[pl.ds(prev * bm, bm), :] = out.astype(bf)
        a0 = uv0_ref[...].astype(bf)
        a1 = uv1_ref[...].astype(bf)
        uvbf_ref[pl.ds(cur * bm, bm), :kh] = a0
        uvbf_ref[pl.ds(cur * bm, bm), kh:] = a1
        acc_ref[...] = (
            jnp.dot(a0, u0_ref[:kh, :], preferred_element_type=f32)
            + jnp.dot(a1, u0_ref[kh:, :], preferred_element_type=f32))

    @pl.when(p == 1)
    def _phase1():
        out = epilogue(w1v_ref, b1v_ref, w2v_ref, b2v_ref)
        u1_ref[pl.ds(prev * bm, bm), :] = out.astype(bf)
        a0 = vu0_ref[...].astype(bf)
        a1 = vu1_ref[...].astype(bf)
        acc_ref[...] = (
            jnp.dot(a0, v1_ref[:kh, :], preferred_element_type=f32)
            + jnp.dot(a1, v1_ref[kh:, :], preferred_element_type=f32))

    @pl.when(p == 2)
    def _phase2():
        o_ref[...] = epilogue(w1u_ref, b1u_ref, w2u_ref, b2u_ref)
        ab = uvbf_ref[pl.ds(cur * bm, bm), :]
        acc_ref[...] = jnp.dot(ab, u1_ref[...], preferred_element_type=f32)


def kernel(u_node_feats, v_node_feats, uv_adj_mat, vu_adj_mat,
           W1_uv, b1_uv, W2_uv, b2_uv, W1_vu, b1_vu, W2_vu, b2_vu):
    bf = jnp.bfloat16
    bm = 256
    n, k = uv_adj_mat.shape
    d = u_node_feats.shape[1]
    kh = k // 2
    nb = n // bm

    u0 = u_node_feats.astype(bf)
    w1u = W1_uv.astype(bf)
    w2u = W2_uv.astype(bf)
    w1v = W1_vu.astype(bf)
    w2v = W2_vu.astype(bf)
    b1u = b1_uv.reshape(1, d)
    b2u = b2_uv.reshape(1, d)
    b1v = b1_vu.reshape(1, d)
    b2v = b2_vu.reshape(1, d)

    uv_idx = lambda p, i: (jnp.where(p == 0, jnp.minimum(i, nb - 1), nb - 1), 0)
    uv_idx1 = lambda p, i: (jnp.where(p == 0, jnp.minimum(i, nb - 1), nb - 1), 1)
    vu_idx = lambda p, i: (jnp.where(p == 1, jnp.minimum(i, nb - 1), 0), 0)
    vu_idx1 = lambda p, i: (jnp.where(p == 1, jnp.minimum(i, nb - 1), 0), 1)
    const = lambda p, i: (0, 0)

    return pl.pallas_call(
        functools.partial(_body, bm=bm, nb=nb, kh=kh),
        grid=(3, nb + 1),
        in_specs=[
            pl.BlockSpec((bm, kh), uv_idx),
            pl.BlockSpec((bm, kh), uv_idx1),
            pl.BlockSpec((bm, kh), vu_idx),
            pl.BlockSpec((bm, kh), vu_idx1),
            pl.BlockSpec((k, d), const),
            pl.BlockSpec((d, d), const),
            pl.BlockSpec((1, d), const),
            pl.BlockSpec((d, d), const),
            pl.BlockSpec((1, d), const),
            pl.BlockSpec((d, d), const),
            pl.BlockSpec((1, d), const),
            pl.BlockSpec((d, d), const),
            pl.BlockSpec((1, d), const),
        ],
        out_specs=pl.BlockSpec(
            (bm, d), lambda p, i: (jnp.where(p == 2, jnp.maximum(i - 1, 0), 0), 0)),
        out_shape=jax.ShapeDtypeStruct((n, d), jnp.float32),
        scratch_shapes=[
            pltpu.VMEM((n, k), bf),
            pltpu.VMEM((n, d), bf),
            pltpu.VMEM((n, d), bf),
            pltpu.VMEM((bm, d), jnp.float32),
        ],
        compiler_params=pltpu.CompilerParams(
            dimension_semantics=("arbitrary", "arbitrary"),
            vmem_limit_bytes=110 * 1024 * 1024,
        ),
    )(uv_adj_mat, uv_adj_mat, vu_adj_mat, vu_adj_mat, u0,
      w1u, b1u, w2u, b2u, w1v, b1v, w2v, b2v)


# speedup vs baseline: 1.0528x; 1.0299x over previous
import jax
import jax.numpy as jnp
from jax.experimental import pallas as pl
from jax.experimental.pallas import tpu as pltpu


def _body(uv0_ref, uv1_ref, vu0_ref, vu1_ref, u0_ref, w1_ref, b1_ref,
          w2_ref, b2_ref, o_ref, uvbf_ref, v1_ref, u1_ref, acc_ref, *,
          bm, nb, kh):
    f32 = jnp.float32
    bf = jnp.bfloat16
    p = pl.program_id(0)
    i = pl.program_id(1)
    prev = jnp.maximum(i - 1, 0)
    cur = jnp.minimum(i, nb - 1)

    def epilogue():
        aggb = acc_ref[...].astype(bf)
        h = jnp.dot(aggb, w1_ref[0], preferred_element_type=f32) + b1_ref[0]
        h = jnp.where(h > 0, h, 0.01 * h)
        hb = h.astype(bf)
        o = jnp.dot(hb, w2_ref[0], preferred_element_type=f32) + b2_ref[0]
        return jnp.where(o > 0, o, 0.01 * o)

    @pl.when(p == 0)
    def _phase0():
        out = epilogue()
        v1_ref[pl.ds(prev * bm, bm), :] = out.astype(bf)
        a0 = uv0_ref[...].astype(bf)
        a1 = uv1_ref[...].astype(bf)
        uvbf_ref[pl.ds(cur * bm, bm), :kh] = a0
        uvbf_ref[pl.ds(cur * bm, bm), kh:] = a1
        acc_ref[...] = (
            jnp.dot(a0, u0_ref[:kh, :], preferred_element_type=f32)
            + jnp.dot(a1, u0_ref[kh:, :], preferred_element_type=f32))

    @pl.when(p == 1)
    def _phase1():
        out = epilogue()
        u1_ref[pl.ds(prev * bm, bm), :] = out.astype(bf)
        a0 = vu0_ref[...].astype(bf)
        a1 = vu1_ref[...].astype(bf)
        acc_ref[...] = (
            jnp.dot(a0, v1_ref[:kh, :], preferred_element_type=f32)
            + jnp.dot(a1, v1_ref[kh:, :], preferred_element_type=f32))

    @pl.when(p == 2)
    def _phase2():
        o_ref[...] = epilogue()
        ab = uvbf_ref[pl.ds(cur * bm, bm), :]
        acc_ref[...] = jnp.dot(ab, u1_ref[...], preferred_element_type=f32)


def kernel(u_node_feats, v_node_feats, uv_adj_mat, vu_adj_mat,
           W1_uv, b1_uv, W2_uv, b2_uv, W1_vu, b1_vu, W2_vu, b2_vu):
    import functools
    bf = jnp.bfloat16
    bm = 256
    n, k = uv_adj_mat.shape
    d = u_node_feats.shape[1]
    kh = k // 2
    nb = n // bm

    u0 = u_node_feats.astype(bf)
    w1s = jnp.stack([W1_uv, W1_vu, W1_uv]).astype(bf)
    w2s = jnp.stack([W2_uv, W2_vu, W2_uv]).astype(bf)
    b1s = jnp.stack([b1_uv, b1_vu, b1_uv]).reshape(3, 1, d)
    b2s = jnp.stack([b2_uv, b2_vu, b2_uv]).reshape(3, 1, d)

    uv_idx = lambda p, i: (jnp.where(p == 0, jnp.minimum(i, nb - 1), nb - 1), 0)
    uv_idx1 = lambda p, i: (jnp.where(p == 0, jnp.minimum(i, nb - 1), nb - 1), 1)
    vu_idx = lambda p, i: (jnp.where(p == 1, jnp.minimum(i, nb - 1), 0), 0)
    vu_idx1 = lambda p, i: (jnp.where(p == 1, jnp.minimum(i, nb - 1), 0), 1)
    const = lambda p, i: (0, 0)
    wsel = lambda p, i: (p, 0, 0)

    return pl.pallas_call(
        functools.partial(_body, bm=bm, nb=nb, kh=kh),
        grid=(3, nb + 1),
        in_specs=[
            pl.BlockSpec((bm, kh), uv_idx),
            pl.BlockSpec((bm, kh), uv_idx1),
            pl.BlockSpec((bm, kh), vu_idx),
            pl.BlockSpec((bm, kh), vu_idx1),
            pl.BlockSpec((k, d), const),
            pl.BlockSpec((1, d, d), wsel),
            pl.BlockSpec((1, 1, d), wsel),
            pl.BlockSpec((1, d, d), wsel),
            pl.BlockSpec((1, 1, d), wsel),
        ],
        out_specs=pl.BlockSpec(
            (bm, d), lambda p, i: (jnp.where(p == 2, jnp.maximum(i - 1, 0), 0), 0)),
        out_shape=jax.ShapeDtypeStruct((n, d), jnp.float32),
        scratch_shapes=[
            pltpu.VMEM((n, k), bf),
            pltpu.VMEM((n, d), bf),
            pltpu.VMEM((n, d), bf),
            pltpu.VMEM((bm, d), jnp.float32),
        ],
        compiler_params=pltpu.CompilerParams(
            dimension_semantics=("arbitrary", "arbitrary"),
            vmem_limit_bytes=110 * 1024 * 1024,
        ),
    )(uv_adj_mat, uv_adj_mat, vu_adj_mat, vu_adj_mat, u0,
      w1s, b1s, w2s, b2s)
